# Initial kernel scaffold; baseline (speedup 1.0000x reference)
import jax
import jax.numpy as jnp
from jax.experimental import pallas as pl


def kernel(xy, tables):
    raise NotImplementedError("write your pallas kernel here")


# speedup vs baseline: 2.3153x; 2.3153x over previous
import math

import jax
import jax.numpy as jnp
from jax import lax
from jax.experimental import pallas as pl
from jax.experimental.pallas import tpu as pltpu
from jax.experimental.pallas import tpu_sc as plsc

L = 16
F = 2
T = 262144
NMIN = 16
NMAX = 131072
_B_GROWTH = math.exp((math.log(NMAX) - math.log(NMIN)) / (L - 1))
_NS = [int(math.floor(NMIN * _B_GROWTH ** l)) for l in range(L)]
_PI2_32 = -1640531535

N = 262144
NC = 2
NSUB = 16
NW = NC * NSUB
PPW = N // NW
BK = 512
NBLK = PPW // BK
VL = 16
GPL = T // 8

_STAGED = [0, 1, 2, 3]
_LOC_CNT = [-(-((_NS[l] + 1) ** 2) // 8) for l in _STAGED]
_LOC_OFF = [sum(_LOC_CNT[:i]) for i in range(len(_STAGED))]
LOCG = sum(_LOC_CNT)


def _corner_indices(xv, yv, off, nl, dense):
    xx = jnp.clip(xv[pl.ds(off, VL)], 0.0, 1.0)
    yy = jnp.clip(yv[pl.ds(off, VL)], 0.0, 1.0)
    px = xx * jnp.float32(nl)
    py = yy * jnp.float32(nl)
    x0 = px.astype(jnp.int32)
    y0 = py.astype(jnp.int32)
    wx = px - x0.astype(jnp.float32)
    wy = py - y0.astype(jnp.float32)
    x1 = jnp.minimum(x0 + 1, nl)
    y1 = jnp.minimum(y0 + 1, nl)
    if dense:
        r00 = x0 * (nl + 1) + y0
        r10 = x1 * (nl + 1) + y0
        r01 = x0 * (nl + 1) + y1
        r11 = x1 * (nl + 1) + y1
    else:
        m = jnp.int32(T - 1)
        h0 = y0 * jnp.int32(_PI2_32)
        h1 = y1 * jnp.int32(_PI2_32)
        r00 = (x0 ^ h0) & m
        r10 = (x1 ^ h0) & m
        r01 = (x0 ^ h1) & m
        r11 = (x1 ^ h1) & m
    return (r00, r10, r01, r11), wx, wy


def _lerp(fs, wx, wy):
    (f00a, f00b), (f10a, f10b), (f01a, f01b), (f11a, f11b) = fs
    fa0 = f00a + wx * (f10a - f00a)
    fa1 = f01a + wx * (f11a - f01a)
    fb0 = f00b + wx * (f10b - f00b)
    fb1 = f01b + wx * (f11b - f01b)
    return fa0 + wy * (fa1 - fa0), fb0 + wy * (fb1 - fb0)


def _body(x_hbm, y_hbm, tab_hbm, out_hbm, xv, yv, wxv, wyv, idxv, subv, rowsv,
          outv, locv, sem):
    c = lax.axis_index("c")
    s = lax.axis_index("s")
    wid = s * NC + c

    lanes = lax.iota(jnp.int32, VL)

    for i, l in enumerate(_STAGED):
        pltpu.sync_copy(
            tab_hbm.at[pl.ds(jnp.int32(l * GPL), _LOC_CNT[i])],
            locv.at[pl.ds(jnp.int32(_LOC_OFF[i]), _LOC_CNT[i])],
        )

    def block(b, carry):
        base = wid * jnp.int32(PPW) + b * jnp.int32(BK)
        pltpu.sync_copy(x_hbm.at[pl.ds(base, BK)], xv)
        pltpu.sync_copy(y_hbm.at[pl.ds(base, BK)], yv)

        for i, l in enumerate(_STAGED):
            nl = _NS[l]
            loc0 = jnp.int32(_LOC_OFF[i])

            def staged_step(j, _, nl=nl, loc0=loc0, l=l):
                off = j * jnp.int32(VL)
                rs, wx, wy = _corner_indices(xv, yv, off, nl, True)
                fs = []
                for r in rs:
                    g = loc0 + (r >> 3)
                    sub = (r & jnp.int32(7)) * 2
                    fs.append((plsc.load_gather(locv, [g, sub]),
                               plsc.load_gather(locv, [g, sub + 1])))
                fa, fb = _lerp(fs, wx, wy)
                rows = off + lanes
                cola = jnp.full((VL,), 2 * l, jnp.int32)
                plsc.store_scatter(outv, [rows, cola], fa)
                plsc.store_scatter(outv, [rows, cola + 1], fb)
                return 0

            lax.fori_loop(jnp.int32(0), jnp.int32(BK // VL), staged_step, 0,
                          unroll=False)

        for l in range(L):
            if l in _STAGED:
                continue
            nl = _NS[l]
            dense = (nl + 1) * (nl + 1) <= T
            gbase = jnp.int32(l * GPL)

            def idx_step(j, _, nl=nl, dense=dense, gbase=gbase):
                off = j * jnp.int32(VL)
                rs, wx, wy = _corner_indices(xv, yv, off, nl, dense)
                wxv[pl.ds(off, VL)] = wx
                wyv[pl.ds(off, VL)] = wy
                for ci, r in enumerate(rs):
                    idxv[pl.ds(jnp.int32(ci * BK) + off, VL)] = gbase + (r >> 3)
                    subv[pl.ds(jnp.int32(ci * BK) + off, VL)] = (r & jnp.int32(7)) * 2
                return 0

            lax.fori_loop(jnp.int32(0), jnp.int32(BK // VL), idx_step, 0,
                          unroll=False)

            pltpu.async_copy(tab_hbm.at[idxv], rowsv, sem).wait()

            def interp_step(j, _, l=l):
                off = j * jnp.int32(VL)
                fs = []
                for ci in range(4):
                    seg = jnp.int32(ci * BK) + off
                    g = seg + lanes
                    sub = subv[pl.ds(seg, VL)]
                    fs.append((plsc.load_gather(rowsv, [g, sub]),
                               plsc.load_gather(rowsv, [g, sub + 1])))
                wx = wxv[pl.ds(off, VL)]
                wy = wyv[pl.ds(off, VL)]
                fa, fb = _lerp(fs, wx, wy)
                rows = off + lanes
                cola = jnp.full((VL,), 2 * l, jnp.int32)
                plsc.store_scatter(outv, [rows, cola], fa)
                plsc.store_scatter(outv, [rows, cola + 1], fb)
                return 0

            lax.fori_loop(jnp.int32(0), jnp.int32(BK // VL), interp_step, 0,
                          unroll=False)

        pltpu.sync_copy(outv, out_hbm.at[pl.ds(base, BK)])
        return carry

    lax.fori_loop(jnp.int32(0), jnp.int32(NBLK), block, 0, unroll=False)


@jax.jit
def _encode(x, y, tab):
    mesh = plsc.VectorSubcoreMesh(
        core_axis_name="c", subcore_axis_name="s", num_cores=NC, num_subcores=NSUB
    )
    return pl.kernel(
        _body,
        out_type=jax.ShapeDtypeStruct((N, 2 * L), jnp.float32),
        mesh=mesh,
        scratch_types=[
            pltpu.VMEM((BK,), jnp.float32),
            pltpu.VMEM((BK,), jnp.float32),
            pltpu.VMEM((BK,), jnp.float32),
            pltpu.VMEM((BK,), jnp.float32),
            pltpu.VMEM((4 * BK,), jnp.int32),
            pltpu.VMEM((4 * BK,), jnp.int32),
            pltpu.VMEM((4 * BK, 16), jnp.float32),
            pltpu.VMEM((BK, 2 * L), jnp.float32),
            pltpu.VMEM((LOCG, 16), jnp.float32),
            pltpu.SemaphoreType.DMA,
        ],
        compiler_params=pltpu.CompilerParams(
            needs_layout_passes=False, use_tc_tiling_on_sc=False
        ),
    )(x, y, tab)


def kernel(xy, tables):
    xy = xy.astype(jnp.float32)
    x = xy[:, 0]
    y = xy[:, 1]
    tab = tables.astype(jnp.float32).reshape(L * T * F // 16, 16)
    return _encode(x, y, tab)


# speedup vs baseline: 12.2577x; 5.2942x over previous
import math

import jax
import jax.numpy as jnp
from jax import lax
from jax.experimental import pallas as pl
from jax.experimental.pallas import tpu as pltpu
from jax.experimental.pallas import tpu_sc as plsc

L = 16
F = 2
T = 262144
NMIN = 16
NMAX = 131072
_B_GROWTH = math.exp((math.log(NMAX) - math.log(NMIN)) / (L - 1))
_NS = [int(math.floor(NMIN * _B_GROWTH ** l)) for l in range(L)]
_PI2_32 = -1640531535

N = 262144
NC = 2
NSUB = 16
NW = NC * NSUB
PPW = N // NW
BK = 512
NBLK = PPW // BK
VL = 16
GPL = T // 8

_STAGED = [0, 1, 2, 3]
_LOC_CNT = [-(-((_NS[l] + 1) ** 2) // 8) for l in _STAGED]
_LOC_OFF = [sum(_LOC_CNT[:i]) for i in range(len(_STAGED))]
LOCG = sum(_LOC_CNT)


def _corner_indices(xv, yv, off, nl, dense):
    xx = jnp.clip(xv[pl.ds(off, VL)], 0.0, 1.0)
    yy = jnp.clip(yv[pl.ds(off, VL)], 0.0, 1.0)
    px = xx * jnp.float32(nl)
    py = yy * jnp.float32(nl)
    x0 = px.astype(jnp.int32)
    y0 = py.astype(jnp.int32)
    wx = px - x0.astype(jnp.float32)
    wy = py - y0.astype(jnp.float32)
    x1 = jnp.minimum(x0 + 1, nl)
    y1 = jnp.minimum(y0 + 1, nl)
    if dense:
        r00 = x0 * (nl + 1) + y0
        r10 = x1 * (nl + 1) + y0
        r01 = x0 * (nl + 1) + y1
        r11 = x1 * (nl + 1) + y1
    else:
        m = jnp.int32(T - 1)
        h0 = y0 * jnp.int32(_PI2_32)
        h1 = y1 * jnp.int32(_PI2_32)
        r00 = (x0 ^ h0) & m
        r10 = (x1 ^ h0) & m
        r01 = (x0 ^ h1) & m
        r11 = (x1 ^ h1) & m
    return (r00, r10, r01, r11), wx, wy


def _lerp(fs, wx, wy):
    (f00a, f00b), (f10a, f10b), (f01a, f01b), (f11a, f11b) = fs
    fa0 = f00a + wx * (f10a - f00a)
    fa1 = f01a + wx * (f11a - f01a)
    fb0 = f00b + wx * (f10b - f00b)
    fb1 = f01b + wx * (f11b - f01b)
    return fa0 + wy * (fa1 - fa0), fb0 + wy * (fb1 - fb0)


def _body(x_hbm, y_hbm, tab_hbm, out_hbm, xv, yv, wxv, wyv, idxv, subv, rowsv,
          outv, locv, sem):
    c = lax.axis_index("c")
    s = lax.axis_index("s")
    wid = s * NC + c

    lanes = lax.iota(jnp.int32, VL)
    tabg = tab_hbm

    for i, l in enumerate(_STAGED):
        pltpu.sync_copy(
            tabg.at[pl.ds(jnp.int32(l * GPL), _LOC_CNT[i])],
            locv.at[pl.ds(jnp.int32(_LOC_OFF[i]), _LOC_CNT[i])],
        )

    def block(b, carry):
        base = wid * jnp.int32(PPW) + b * jnp.int32(BK)
        pltpu.sync_copy(x_hbm.at[pl.ds(base, BK)], xv)
        pltpu.sync_copy(y_hbm.at[pl.ds(base, BK)], yv)

        for i, l in enumerate(_STAGED):
            nl = _NS[l]
            loc0 = jnp.int32(_LOC_OFF[i])

            def staged_step(j, _, nl=nl, loc0=loc0, l=l):
                off = j * jnp.int32(VL)
                rs, wx, wy = _corner_indices(xv, yv, off, nl, True)
                fs = []
                for r in rs:
                    g = loc0 + (r >> 3)
                    sub = (r & jnp.int32(7)) * 2
                    fs.append((plsc.load_gather(locv, [g, sub]),
                               plsc.load_gather(locv, [g, sub + 1])))
                fa, fb = _lerp(fs, wx, wy)
                flat = (off + lanes) * jnp.int32(2 * L) + jnp.int32(2 * l)
                plsc.store_scatter(outv, [flat], fa)
                plsc.store_scatter(outv, [flat + 1], fb)
                return 0

            lax.fori_loop(jnp.int32(0), jnp.int32(BK // VL), staged_step, 0,
                          unroll=False)

        for l in range(L):
            if l in _STAGED:
                continue
            nl = _NS[l]
            dense = (nl + 1) * (nl + 1) <= T
            gbase = jnp.int32(l * GPL)

            def idx_step(j, _, nl=nl, dense=dense, gbase=gbase):
                off = j * jnp.int32(VL)
                rs, wx, wy = _corner_indices(xv, yv, off, nl, dense)
                wxv[pl.ds(off, VL)] = wx
                wyv[pl.ds(off, VL)] = wy
                for ci, r in enumerate(rs):
                    idxv[pl.ds(jnp.int32(ci * BK) + off, VL)] = gbase + (r >> 3)
                    subv[pl.ds(jnp.int32(ci * BK) + off, VL)] = (r & jnp.int32(7)) * 2
                return 0

            lax.fori_loop(jnp.int32(0), jnp.int32(BK // VL), idx_step, 0,
                          unroll=False)

            pltpu.async_copy(tabg.at[idxv], rowsv, sem).wait()

            def interp_step(j, _, l=l):
                off = j * jnp.int32(VL)
                fs = []
                for ci in range(4):
                    seg = jnp.int32(ci * BK) + off
                    g = seg + lanes
                    sub = subv[pl.ds(seg, VL)]
                    fs.append((plsc.load_gather(rowsv, [g, sub]),
                               plsc.load_gather(rowsv, [g, sub + 1])))
                wx = wxv[pl.ds(off, VL)]
                wy = wyv[pl.ds(off, VL)]
                fa, fb = _lerp(fs, wx, wy)
                flat = (off + lanes) * jnp.int32(2 * L) + jnp.int32(2 * l)
                plsc.store_scatter(outv, [flat], fa)
                plsc.store_scatter(outv, [flat + 1], fb)
                return 0

            lax.fori_loop(jnp.int32(0), jnp.int32(BK // VL), interp_step, 0,
                          unroll=False)

        pltpu.sync_copy(outv, out_hbm.at[pl.ds(base * jnp.int32(2 * L), BK * 2 * L)])
        return carry

    lax.fori_loop(jnp.int32(0), jnp.int32(NBLK), block, 0, unroll=False)


def _relayout_body(tabp_hbm, tabi_hbm, inv, outv):
    c = lax.axis_index("c")
    s = lax.axis_index("s")
    wid = s * NC + c
    lanes = lax.iota(jnp.int32, VL)
    sub_hi = (lanes >> 3) << 4
    sub_lo = (lanes & jnp.int32(7)) << 1

    def lvl(l, carry):
        in0 = (l * jnp.int32(4096) + wid * jnp.int32(128)) * jnp.int32(128)
        out0 = (l * jnp.int32(32768) + wid * jnp.int32(1024)) * jnp.int32(16)
        pltpu.sync_copy(tabp_hbm.at[pl.ds(in0, 16384)], inv)

        def ustep(u, _):
            ub = u * jnp.int32(256)
            for k in range(8):
                f0 = inv[pl.ds(ub + 16 * k, VL)]
                f1 = inv[pl.ds(ub + 128 + 16 * k, VL)]
                dst = ub + jnp.int32(32 * k) + sub_hi + sub_lo
                plsc.store_scatter(outv, [dst], f0)
                plsc.store_scatter(outv, [dst + 1], f1)
            return 0

        lax.fori_loop(jnp.int32(0), jnp.int32(64), ustep, 0, unroll=False)
        pltpu.sync_copy(outv, tabi_hbm.at[pl.ds(out0, 16384)])
        return carry

    lax.fori_loop(jnp.int32(0), jnp.int32(L), lvl, 0, unroll=False)


@jax.jit
def _relayout(tabp):
    mesh = plsc.VectorSubcoreMesh(
        core_axis_name="c", subcore_axis_name="s", num_cores=NC, num_subcores=NSUB
    )
    return pl.kernel(
        _relayout_body,
        out_type=jax.ShapeDtypeStruct((L * T * F,), jnp.float32),
        mesh=mesh,
        scratch_types=[
            pltpu.VMEM((16384,), jnp.float32),
            pltpu.VMEM((16384,), jnp.float32),
        ],
        compiler_params=pltpu.CompilerParams(
            needs_layout_passes=False, use_tc_tiling_on_sc=False
        ),
    )(tabp)


@jax.jit
def _encode(x, y, tab):
    mesh = plsc.VectorSubcoreMesh(
        core_axis_name="c", subcore_axis_name="s", num_cores=NC, num_subcores=NSUB
    )
    return pl.kernel(
        _body,
        out_type=jax.ShapeDtypeStruct((N * 2 * L,), jnp.float32),
        mesh=mesh,
        scratch_types=[
            pltpu.VMEM((BK,), jnp.float32),
            pltpu.VMEM((BK,), jnp.float32),
            pltpu.VMEM((BK,), jnp.float32),
            pltpu.VMEM((BK,), jnp.float32),
            pltpu.VMEM((4 * BK,), jnp.int32),
            pltpu.VMEM((4 * BK,), jnp.int32),
            pltpu.VMEM((4 * BK, 16), jnp.float32),
            pltpu.VMEM((BK * 2 * L,), jnp.float32),
            pltpu.VMEM((LOCG, 16), jnp.float32),
            pltpu.SemaphoreType.DMA,
        ],
        compiler_params=pltpu.CompilerParams(
            needs_layout_passes=False, use_tc_tiling_on_sc=False
        ),
    )(x, y, tab)


def kernel(xy, tables):
    xy = xy.astype(jnp.float32)
    x = xy[:, 0]
    y = xy[:, 1]
    tabp = (
        tables.astype(jnp.float32)
        .reshape(L, T // 128, 128, F)
        .transpose(0, 1, 3, 2)
        .reshape(L * T * F)
    )
    tabi = _relayout(tabp).reshape(L * T * F // 16, 16)
    return _encode(x, y, tabi).reshape(N, 2 * L)


# speedup vs baseline: 18.6011x; 1.5175x over previous
import math

import jax
import jax.numpy as jnp
from jax import lax
from jax.experimental import pallas as pl
from jax.experimental.pallas import tpu as pltpu
from jax.experimental.pallas import tpu_sc as plsc

L = 16
F = 2
T = 262144
NMIN = 16
NMAX = 131072
_B_GROWTH = math.exp((math.log(NMAX) - math.log(NMIN)) / (L - 1))
_NS = [int(math.floor(NMIN * _B_GROWTH ** l)) for l in range(L)]
_PI2_32 = -1640531535

N = 262144
NC = 2
NSUB = 16
NW = NC * NSUB
PPW = N // NW
BK = 512
NBLK = PPW // BK
VL = 16
GPL = T // 8

_STAGED = [0, 1, 2, 3]
_LOC_CNT = [-(-((_NS[l] + 1) ** 2) // 8) for l in _STAGED]
_LOC_OFF = [sum(_LOC_CNT[:i]) for i in range(len(_STAGED))]
LOCG = sum(_LOC_CNT)


def _corner_indices(xv, yv, off, nl, dense):
    xx = jnp.clip(xv[pl.ds(off, VL)], 0.0, 1.0)
    yy = jnp.clip(yv[pl.ds(off, VL)], 0.0, 1.0)
    px = xx * jnp.float32(nl)
    py = yy * jnp.float32(nl)
    x0 = px.astype(jnp.int32)
    y0 = py.astype(jnp.int32)
    wx = px - x0.astype(jnp.float32)
    wy = py - y0.astype(jnp.float32)
    x1 = jnp.minimum(x0 + 1, nl)
    y1 = jnp.minimum(y0 + 1, nl)
    if dense:
        r00 = x0 * (nl + 1) + y0
        r10 = x1 * (nl + 1) + y0
        r01 = x0 * (nl + 1) + y1
        r11 = x1 * (nl + 1) + y1
    else:
        m = jnp.int32(T - 1)
        h0 = y0 * jnp.int32(_PI2_32)
        h1 = y1 * jnp.int32(_PI2_32)
        r00 = (x0 ^ h0) & m
        r10 = (x1 ^ h0) & m
        r01 = (x0 ^ h1) & m
        r11 = (x1 ^ h1) & m
    return (r00, r10, r01, r11), wx, wy


def _lerp(fs, wx, wy):
    (f00a, f00b), (f10a, f10b), (f01a, f01b), (f11a, f11b) = fs
    fa0 = f00a + wx * (f10a - f00a)
    fa1 = f01a + wx * (f11a - f01a)
    fb0 = f00b + wx * (f10b - f00b)
    fb1 = f01b + wx * (f11b - f01b)
    return fa0 + wy * (fa1 - fa0), fb0 + wy * (fb1 - fb0)


def _body(x_hbm, y_hbm, tab_hbm, out_hbm, xv, yv, wxv0, wyv0, wxv1, wyv1,
          idxv0, subv0, idxv1, subv1, rowsv0, rowsv1, outv, locv, sem0, sem1):
    c = lax.axis_index("c")
    s = lax.axis_index("s")
    wid = s * NC + c

    lanes = lax.iota(jnp.int32, VL)
    tabg = tab_hbm
    bufs = [(wxv0, wyv0, idxv0, subv0, rowsv0, sem0),
            (wxv1, wyv1, idxv1, subv1, rowsv1, sem1)]
    HL = [l for l in range(L) if l not in _STAGED]

    for i, l in enumerate(_STAGED):
        pltpu.sync_copy(
            tabg.at[pl.ds(jnp.int32(l * GPL), _LOC_CNT[i])],
            locv.at[pl.ds(jnp.int32(_LOC_OFF[i]), _LOC_CNT[i])],
        )

    def _idx_phase(l, wxv, wyv, idxv, subv):
        nl = _NS[l]
        dense = (nl + 1) * (nl + 1) <= T
        gbase = jnp.int32(l * GPL)

        def idx_step(j, _):
            off = j * jnp.int32(VL)
            rs, wx, wy = _corner_indices(xv, yv, off, nl, dense)
            wxv[pl.ds(off, VL)] = wx
            wyv[pl.ds(off, VL)] = wy
            for ci, r in enumerate(rs):
                idxv[pl.ds(jnp.int32(ci * BK) + off, VL)] = gbase + (r >> 3)
                subv[pl.ds(jnp.int32(ci * BK) + off, VL)] = (r & jnp.int32(7)) * 2
            return 0

        lax.fori_loop(jnp.int32(0), jnp.int32(BK // VL), idx_step, 0,
                      unroll=False)

    def _interp_phase(l, wxv, wyv, subv, rowsv):
        def interp_step(j, _):
            off = j * jnp.int32(VL)
            fs = []
            for ci in range(4):
                seg = jnp.int32(ci * BK) + off
                g = seg + lanes
                sub = subv[pl.ds(seg, VL)]
                fs.append((plsc.load_gather(rowsv, [g, sub]),
                           plsc.load_gather(rowsv, [g, sub + 1])))
            wx = wxv[pl.ds(off, VL)]
            wy = wyv[pl.ds(off, VL)]
            fa, fb = _lerp(fs, wx, wy)
            flat = (off + lanes) * jnp.int32(2 * L) + jnp.int32(2 * l)
            plsc.store_scatter(outv, [flat], fa)
            plsc.store_scatter(outv, [flat + 1], fb)
            return 0

        lax.fori_loop(jnp.int32(0), jnp.int32(BK // VL), interp_step, 0,
                      unroll=False)

    def block(b, carry):
        base = wid * jnp.int32(PPW) + b * jnp.int32(BK)
        pltpu.sync_copy(x_hbm.at[pl.ds(base, BK)], xv)
        pltpu.sync_copy(y_hbm.at[pl.ds(base, BK)], yv)

        wxv, wyv, idxv, subv, rowsv, sem = bufs[0]
        _idx_phase(HL[0], wxv, wyv, idxv, subv)
        copies = [None, None]
        copies[0] = pltpu.async_copy(tabg.at[idxv], rowsv, sem)

        for i, l in enumerate(_STAGED):
            nl = _NS[l]
            loc0 = jnp.int32(_LOC_OFF[i])

            def staged_step(j, _, nl=nl, loc0=loc0, l=l):
                off = j * jnp.int32(VL)
                rs, wx, wy = _corner_indices(xv, yv, off, nl, True)
                fs = []
                for r in rs:
                    g = loc0 + (r >> 3)
                    sub = (r & jnp.int32(7)) * 2
                    fs.append((plsc.load_gather(locv, [g, sub]),
                               plsc.load_gather(locv, [g, sub + 1])))
                fa, fb = _lerp(fs, wx, wy)
                flat = (off + lanes) * jnp.int32(2 * L) + jnp.int32(2 * l)
                plsc.store_scatter(outv, [flat], fa)
                plsc.store_scatter(outv, [flat + 1], fb)
                return 0

            lax.fori_loop(jnp.int32(0), jnp.int32(BK // VL), staged_step, 0,
                          unroll=False)

        for i, l in enumerate(HL):
            if i + 1 < len(HL):
                wxn, wyn, idxn, subn, rowsn, semn = bufs[(i + 1) % 2]
                _idx_phase(HL[i + 1], wxn, wyn, idxn, subn)
                copies[(i + 1) % 2] = pltpu.async_copy(
                    tabg.at[idxn], rowsn, semn
                )
            wxv, wyv, idxv, subv, rowsv, sem = bufs[i % 2]
            copies[i % 2].wait()
            _interp_phase(l, wxv, wyv, subv, rowsv)

        pltpu.sync_copy(outv, out_hbm.at[pl.ds(base * jnp.int32(2 * L), BK * 2 * L)])
        return carry

    lax.fori_loop(jnp.int32(0), jnp.int32(NBLK), block, 0, unroll=False)


def _relayout_body(tabp_hbm, tabi_hbm, inv, outv):
    c = lax.axis_index("c")
    s = lax.axis_index("s")
    wid = s * NC + c
    lanes = lax.iota(jnp.int32, VL)
    sub_hi = (lanes >> 3) << 4
    sub_lo = (lanes & jnp.int32(7)) << 1

    def lvl(l, carry):
        in0 = (l * jnp.int32(4096) + wid * jnp.int32(128)) * jnp.int32(128)
        out0 = (l * jnp.int32(32768) + wid * jnp.int32(1024)) * jnp.int32(16)
        pltpu.sync_copy(tabp_hbm.at[pl.ds(in0, 16384)], inv)

        def ustep(u, _):
            ub = u * jnp.int32(256)
            for k in range(8):
                f0 = inv[pl.ds(ub + 16 * k, VL)]
                f1 = inv[pl.ds(ub + 128 + 16 * k, VL)]
                dst = ub + jnp.int32(32 * k) + sub_hi + sub_lo
                plsc.store_scatter(outv, [dst], f0)
                plsc.store_scatter(outv, [dst + 1], f1)
            return 0

        lax.fori_loop(jnp.int32(0), jnp.int32(64), ustep, 0, unroll=False)
        pltpu.sync_copy(outv, tabi_hbm.at[pl.ds(out0, 16384)])
        return carry

    lax.fori_loop(jnp.int32(0), jnp.int32(L), lvl, 0, unroll=False)


@jax.jit
def _relayout(tabp):
    mesh = plsc.VectorSubcoreMesh(
        core_axis_name="c", subcore_axis_name="s", num_cores=NC, num_subcores=NSUB
    )
    return pl.kernel(
        _relayout_body,
        out_type=jax.ShapeDtypeStruct((L * T * F,), jnp.float32),
        mesh=mesh,
        scratch_types=[
            pltpu.VMEM((16384,), jnp.float32),
            pltpu.VMEM((16384,), jnp.float32),
        ],
        compiler_params=pltpu.CompilerParams(
            needs_layout_passes=False, use_tc_tiling_on_sc=False
        ),
    )(tabp)


@jax.jit
def _encode(x, y, tab):
    mesh = plsc.VectorSubcoreMesh(
        core_axis_name="c", subcore_axis_name="s", num_cores=NC, num_subcores=NSUB
    )
    return pl.kernel(
        _body,
        out_type=jax.ShapeDtypeStruct((N * 2 * L,), jnp.float32),
        mesh=mesh,
        scratch_types=[
            pltpu.VMEM((BK,), jnp.float32),
            pltpu.VMEM((BK,), jnp.float32),
            pltpu.VMEM((BK,), jnp.float32),
            pltpu.VMEM((BK,), jnp.float32),
            pltpu.VMEM((BK,), jnp.float32),
            pltpu.VMEM((BK,), jnp.float32),
            pltpu.VMEM((4 * BK,), jnp.int32),
            pltpu.VMEM((4 * BK,), jnp.int32),
            pltpu.VMEM((4 * BK,), jnp.int32),
            pltpu.VMEM((4 * BK,), jnp.int32),
            pltpu.VMEM((4 * BK, 16), jnp.float32),
            pltpu.VMEM((4 * BK, 16), jnp.float32),
            pltpu.VMEM((BK * 2 * L,), jnp.float32),
            pltpu.VMEM((LOCG, 16), jnp.float32),
            pltpu.SemaphoreType.DMA,
            pltpu.SemaphoreType.DMA,
        ],
        compiler_params=pltpu.CompilerParams(
            needs_layout_passes=False, use_tc_tiling_on_sc=False
        ),
    )(x, y, tab)


def kernel(xy, tables):
    xy = xy.astype(jnp.float32)
    x = xy[:, 0]
    y = xy[:, 1]
    tabp = (
        tables.astype(jnp.float32)
        .reshape(L, T // 128, 128, F)
        .transpose(0, 1, 3, 2)
        .reshape(L * T * F)
    )
    tabi = _relayout(tabp).reshape(L * T * F // 16, 16)
    return _encode(x, y, tabi).reshape(N, 2 * L)


# speedup vs baseline: 23.5885x; 1.2681x over previous
import math

import jax
import jax.numpy as jnp
from jax import lax
from jax.experimental import pallas as pl
from jax.experimental.pallas import tpu as pltpu
from jax.experimental.pallas import tpu_sc as plsc

L = 16
F = 2
T = 262144
NMIN = 16
NMAX = 131072
_B_GROWTH = math.exp((math.log(NMAX) - math.log(NMIN)) / (L - 1))
_NS = [int(math.floor(NMIN * _B_GROWTH ** l)) for l in range(L)]
_PI2_32 = -1640531535

N = 262144
NC = 2
NSUB = 16
NW = NC * NSUB
PPW = N // NW
BK = 512
NBLK = PPW // BK
VL = 16
GPL = T // 8

_STAGED = [0, 1, 2, 3]
_LOC_CNT = [-(-((_NS[l] + 1) ** 2) // 8) for l in _STAGED]
_LOC_OFF = [sum(_LOC_CNT[:i]) for i in range(len(_STAGED))]
LOCG = sum(_LOC_CNT)


def _corner_indices(xyv, off, nl, dense):
    xoff = ((off >> 7) << 8) + (off & jnp.int32(127))
    xx = jnp.clip(xyv[pl.ds(xoff, VL)], 0.0, 1.0)
    yy = jnp.clip(xyv[pl.ds(xoff + 128, VL)], 0.0, 1.0)
    px = xx * jnp.float32(nl)
    py = yy * jnp.float32(nl)
    x0 = px.astype(jnp.int32)
    y0 = py.astype(jnp.int32)
    wx = px - x0.astype(jnp.float32)
    wy = py - y0.astype(jnp.float32)
    x1 = jnp.minimum(x0 + 1, nl)
    y1 = jnp.minimum(y0 + 1, nl)
    if dense:
        r00 = x0 * (nl + 1) + y0
        r10 = x1 * (nl + 1) + y0
        r01 = x0 * (nl + 1) + y1
        r11 = x1 * (nl + 1) + y1
    else:
        m = jnp.int32(T - 1)
        h0 = y0 * jnp.int32(_PI2_32)
        h1 = y1 * jnp.int32(_PI2_32)
        r00 = (x0 ^ h0) & m
        r10 = (x1 ^ h0) & m
        r01 = (x0 ^ h1) & m
        r11 = (x1 ^ h1) & m
    return (r00, r10, r01, r11), wx, wy


def _lerp(fs, wx, wy):
    (f00a, f00b), (f10a, f10b), (f01a, f01b), (f11a, f11b) = fs
    fa0 = f00a + wx * (f10a - f00a)
    fa1 = f01a + wx * (f11a - f01a)
    fb0 = f00b + wx * (f10b - f00b)
    fb1 = f01b + wx * (f11b - f01b)
    return fa0 + wy * (fa1 - fa0), fb0 + wy * (fb1 - fb0)


def _body(xy_hbm, tab_hbm, out_hbm, xyv, wxv0, wyv0, wxv1, wyv1,
          idxv0, subv0, idxv1, subv1, rowsv0, rowsv1, outv, locv, sem0, sem1):
    c = lax.axis_index("c")
    s = lax.axis_index("s")
    wid = s * NC + c

    lanes = lax.iota(jnp.int32, VL)
    tabg = tab_hbm
    bufs = [(wxv0, wyv0, idxv0, subv0, rowsv0, sem0),
            (wxv1, wyv1, idxv1, subv1, rowsv1, sem1)]
    HL = [l for l in range(L) if l not in _STAGED]

    for i, l in enumerate(_STAGED):
        pltpu.sync_copy(
            tabg.at[pl.ds(jnp.int32(l * GPL), _LOC_CNT[i])],
            locv.at[pl.ds(jnp.int32(_LOC_OFF[i]), _LOC_CNT[i])],
        )

    def _idx_phase(l, wxv, wyv, idxv, subv):
        nl = _NS[l]
        dense = (nl + 1) * (nl + 1) <= T
        gbase = jnp.int32(l * GPL)

        def idx_step(j, _):
            off = j * jnp.int32(VL)
            rs, wx, wy = _corner_indices(xyv, off, nl, dense)
            wxv[pl.ds(off, VL)] = wx
            wyv[pl.ds(off, VL)] = wy
            for ci, r in enumerate(rs):
                idxv[pl.ds(jnp.int32(ci * BK) + off, VL)] = gbase + (r >> 3)
                subv[pl.ds(jnp.int32(ci * BK) + off, VL)] = (r & jnp.int32(7)) * 2
            return 0

        lax.fori_loop(jnp.int32(0), jnp.int32(BK // VL), idx_step, 0,
                      unroll=False)

    def _interp_phase(l, wxv, wyv, subv, rowsv):
        def interp_step(j, _):
            off = j * jnp.int32(VL)
            fs = []
            for ci in range(4):
                seg = jnp.int32(ci * BK) + off
                g = seg + lanes
                sub = subv[pl.ds(seg, VL)]
                fs.append((plsc.load_gather(rowsv, [g, sub]),
                           plsc.load_gather(rowsv, [g, sub + 1])))
            wx = wxv[pl.ds(off, VL)]
            wy = wyv[pl.ds(off, VL)]
            fa, fb = _lerp(fs, wx, wy)
            cl = 2 * l
            flat = (jnp.int32((cl >> 3) * 4096 + (cl & 7) * 128)
                    + (off >> 7) * jnp.int32(1024) + (off & jnp.int32(127)) + lanes)
            plsc.store_scatter(outv, [flat], fa)
            plsc.store_scatter(outv, [flat + 128], fb)
            return 0

        lax.fori_loop(jnp.int32(0), jnp.int32(BK // VL), interp_step, 0,
                      unroll=False)

    def block(b, carry):
        base = wid * jnp.int32(PPW) + b * jnp.int32(BK)
        pltpu.sync_copy(xy_hbm.at[pl.ds(base * 2, 2 * BK)], xyv)

        wxv, wyv, idxv, subv, rowsv, sem = bufs[0]
        _idx_phase(HL[0], wxv, wyv, idxv, subv)
        copies = [None, None]
        copies[0] = pltpu.async_copy(tabg.at[idxv], rowsv, sem)

        for i, l in enumerate(_STAGED):
            nl = _NS[l]
            loc0 = jnp.int32(_LOC_OFF[i])

            def staged_step(j, _, nl=nl, loc0=loc0, l=l):
                off = j * jnp.int32(VL)
                rs, wx, wy = _corner_indices(xyv, off, nl, True)
                fs = []
                for r in rs:
                    g = loc0 + (r >> 3)
                    sub = (r & jnp.int32(7)) * 2
                    fs.append((plsc.load_gather(locv, [g, sub]),
                               plsc.load_gather(locv, [g, sub + 1])))
                fa, fb = _lerp(fs, wx, wy)
                cl = 2 * l
                flat = (jnp.int32((cl >> 3) * 4096 + (cl & 7) * 128)
                        + (off >> 7) * jnp.int32(1024)
                        + (off & jnp.int32(127)) + lanes)
                plsc.store_scatter(outv, [flat], fa)
                plsc.store_scatter(outv, [flat + 128], fb)
                return 0

            lax.fori_loop(jnp.int32(0), jnp.int32(BK // VL), staged_step, 0,
                          unroll=False)

        for i, l in enumerate(HL):
            if i + 1 < len(HL):
                wxn, wyn, idxn, subn, rowsn, semn = bufs[(i + 1) % 2]
                _idx_phase(HL[i + 1], wxn, wyn, idxn, subn)
                copies[(i + 1) % 2] = pltpu.async_copy(
                    tabg.at[idxn], rowsn, semn
                )
            wxv, wyv, idxv, subv, rowsv, sem = bufs[i % 2]
            copies[i % 2].wait()
            _interp_phase(l, wxv, wyv, subv, rowsv)

        for tr in range(4):
            pltpu.sync_copy(
                outv.at[pl.ds(jnp.int32(tr * 4096), 4096)],
                out_hbm.at[pl.ds(jnp.int32(tr * 2097152) + base * 8, 4096)],
            )
        return carry

    lax.fori_loop(jnp.int32(0), jnp.int32(NBLK), block, 0, unroll=False)


def _relayout_body(tabp_hbm, tabi_hbm, inv, outv):
    c = lax.axis_index("c")
    s = lax.axis_index("s")
    wid = s * NC + c
    lanes = lax.iota(jnp.int32, VL)
    sub_hi = (lanes >> 3) << 4
    sub_lo = (lanes & jnp.int32(7)) << 1

    def lvl(l, carry):
        in0 = (l * jnp.int32(4096) + wid * jnp.int32(128)) * jnp.int32(128)
        out0 = (l * jnp.int32(32768) + wid * jnp.int32(1024)) * jnp.int32(16)
        pltpu.sync_copy(tabp_hbm.at[pl.ds(in0, 16384)], inv)

        def ustep(u, _):
            ub = u * jnp.int32(256)
            for k in range(8):
                f0 = inv[pl.ds(ub + 16 * k, VL)]
                f1 = inv[pl.ds(ub + 128 + 16 * k, VL)]
                dst = ub + jnp.int32(32 * k) + sub_hi + sub_lo
                plsc.store_scatter(outv, [dst], f0)
                plsc.store_scatter(outv, [dst + 1], f1)
            return 0

        lax.fori_loop(jnp.int32(0), jnp.int32(64), ustep, 0, unroll=False)
        pltpu.sync_copy(outv, tabi_hbm.at[pl.ds(out0, 16384)])
        return carry

    lax.fori_loop(jnp.int32(0), jnp.int32(L), lvl, 0, unroll=False)


@jax.jit
def _relayout(tabp):
    mesh = plsc.VectorSubcoreMesh(
        core_axis_name="c", subcore_axis_name="s", num_cores=NC, num_subcores=NSUB
    )
    return pl.kernel(
        _relayout_body,
        out_type=jax.ShapeDtypeStruct((L * T * F,), jnp.float32),
        mesh=mesh,
        scratch_types=[
            pltpu.VMEM((16384,), jnp.float32),
            pltpu.VMEM((16384,), jnp.float32),
        ],
        compiler_params=pltpu.CompilerParams(
            needs_layout_passes=False, use_tc_tiling_on_sc=False
        ),
    )(tabp)


@jax.jit
def _encode(xyflat, tab):
    mesh = plsc.VectorSubcoreMesh(
        core_axis_name="c", subcore_axis_name="s", num_cores=NC, num_subcores=NSUB
    )
    return pl.kernel(
        _body,
        out_type=jax.ShapeDtypeStruct((N * 2 * L,), jnp.float32),
        mesh=mesh,
        scratch_types=[
            pltpu.VMEM((2 * BK,), jnp.float32),
            pltpu.VMEM((BK,), jnp.float32),
            pltpu.VMEM((BK,), jnp.float32),
            pltpu.VMEM((BK,), jnp.float32),
            pltpu.VMEM((BK,), jnp.float32),
            pltpu.VMEM((4 * BK,), jnp.int32),
            pltpu.VMEM((4 * BK,), jnp.int32),
            pltpu.VMEM((4 * BK,), jnp.int32),
            pltpu.VMEM((4 * BK,), jnp.int32),
            pltpu.VMEM((4 * BK, 16), jnp.float32),
            pltpu.VMEM((4 * BK, 16), jnp.float32),
            pltpu.VMEM((BK * 2 * L,), jnp.float32),
            pltpu.VMEM((LOCG, 16), jnp.float32),
            pltpu.SemaphoreType.DMA,
            pltpu.SemaphoreType.DMA,
        ],
        compiler_params=pltpu.CompilerParams(
            needs_layout_passes=False, use_tc_tiling_on_sc=False
        ),
    )(xyflat, tab)


def kernel(xy, tables):
    xyflat = (
        xy.astype(jnp.float32)
        .reshape(N // 128, 128, 2)
        .transpose(0, 2, 1)
        .reshape(2 * N)
    )
    tabp = (
        tables.astype(jnp.float32)
        .reshape(L, T // 128, 128, F)
        .transpose(0, 1, 3, 2)
        .reshape(L * T * F)
    )
    tabi = _relayout(tabp).reshape(L * T * F // 16, 16)
    out = _encode(xyflat, tabi)
    return (
        out.reshape(4, N // 128, 8, 128)
        .transpose(1, 3, 0, 2)
        .reshape(N, 2 * L)
    )
